# baseline (speedup 1.0000x reference)
import jax
import jax.numpy as jnp
from jax import lax
from jax.experimental import pallas as pl
from jax.experimental.pallas import tpu as pltpu
from jax.experimental.pallas import tpu_sc as plsc

N_NODES = 10000
N_PAD = 10240
RPT = N_PAD // 16
TAIL = N_NODES - RPT * 15
D_FEAT = 128
D_Q = 32
NQ = 4
N_CLASSES = 64
NC = 2
NS = 16
LANE = 128
_NBUF = 4

_MESH = plsc.VectorSubcoreMesh(core_axis_name="c", subcore_axis_name="s")
_SC_PARAMS = pltpu.CompilerParams(use_tc_tiling_on_sc=False,
                                  needs_layout_passes=False)


def _splat(vec_ref, r):
    idx = jnp.zeros((16,), jnp.int32) + r
    return plsc.load_gather(vec_ref, [idx])


def _deg_body(dst_hbm, out_hbm, idx_v, ones_v, zeros_v, deg_sh):
    c = lax.axis_index("c")
    s = lax.axis_index("s")
    w = c * NS + s
    t_rows = dst_hbm.shape[1]

    for i in range(LANE // 16):
        ones_v[pl.ds(i * 16, 16)] = jnp.ones((16,), jnp.float32)
    for i in range(RPT // 16):
        zeros_v[pl.ds(i * 16, 16)] = jnp.zeros((16,), jnp.float32)

    pltpu.sync_copy(zeros_v, deg_sh.at[pl.ds(s * RPT, RPT)])
    pltpu.sync_copy(dst_hbm.at[w], idx_v)
    plsc.subcore_barrier()

    def body(j, carry):
        pltpu.sync_copy(ones_v, deg_sh.at[idx_v.at[j]], add=True)
        return carry

    lax.fori_loop(0, t_rows, body, 0)
    plsc.subcore_barrier()

    pltpu.sync_copy(deg_sh.at[pl.ds(s * RPT, RPT)],
                    out_hbm.at[c, pl.ds(s * RPT, RPT)])


def _deg_call(dst_p):
    t_rows = dst_p.shape[1]
    f = pl.kernel(
        _deg_body,
        out_type=jax.ShapeDtypeStruct((NC, N_PAD), jnp.float32),
        mesh=_MESH,
        scratch_types=[
            pltpu.VMEM((t_rows, LANE), jnp.int32),
            pltpu.VMEM((LANE,), jnp.float32),
            pltpu.VMEM((RPT,), jnp.float32),
            pltpu.VMEM_SHARED((N_PAD,), jnp.float32),
        ],
        compiler_params=_SC_PARAMS,
    )
    return f(dst_p)


def _prop_body(a_hbm, b_hbm, p_hbm, e_hbm, f_hbm, src_hbm, dst_hbm, out_hbm,
               sidx, didx, rows_v, zbuf, a_s, b_s, p_v, e_v, f_v,
               gsems, ssems, g_sh, acc_sh):
    c = lax.axis_index("c")
    s = lax.axis_index("s")
    t_rows = src_hbm.shape[1]
    n_groups = t_rows // _NBUF
    row0 = s * RPT

    def zrow(r, carry):
        for i in range(D_Q // 16):
            zbuf[r, pl.ds(i * 16, 16)] = jnp.zeros((16,), jnp.float32)
        return carry

    lax.fori_loop(0, LANE, zrow, 0)

    pltpu.sync_copy(src_hbm.at[s], sidx)
    pltpu.sync_copy(dst_hbm.at[s], didx)
    pltpu.sync_copy(p_hbm.at[pl.ds(row0, RPT)], p_v)
    pltpu.sync_copy(e_hbm.at[pl.ds(row0, RPT)], e_v)
    pltpu.sync_copy(f_hbm.at[pl.ds(row0, RPT)], f_v)

    def g_issue(j, b):
        pltpu.async_copy(g_sh.at[sidx.at[j]], rows_v.at[b], gsems.at[b])

    def g_wait(j, b):
        pltpu.make_async_copy(
            g_sh.at[sidx.at[j]], rows_v.at[b], gsems.at[b]).wait()

    def s_issue(j, b):
        pltpu.async_copy(rows_v.at[b], acc_sh.at[didx.at[j]],
                         ssems.at[b], add=True)

    def s_wait(j, b):
        pltpu.make_async_copy(
            rows_v.at[b], acc_sh.at[didx.at[j]], ssems.at[b]).wait()

    def edge_pipeline():
        for b in range(_NBUF):
            g_issue(b, b)

        def group(g, carry):
            for b in range(_NBUF):
                j = g * _NBUF + b
                g_wait(j, b)
                s_issue(j, b)
                pb = (b - 1) % _NBUF
                if b == 0:
                    @pl.when(g > 0)
                    def _():
                        s_wait(j - 1, pb)
                        g_issue(j - 1 + _NBUF, pb)
                else:
                    s_wait(j - 1, pb)

                    @pl.when(g < n_groups - 1)
                    def _():
                        g_issue(j - 1 + _NBUF, pb)
            return carry

        lax.fori_loop(0, n_groups, group, 0)
        s_wait(t_rows - 1, (t_rows - 1) % _NBUF)

    def stage_chunk(r0):
        def srow(r, carry):
            d = _splat(p_v, r0 + r)
            for i in range(D_Q // 16):
                sl = pl.ds(i * 16, 16)
                b_s[r, sl] = (a_s[r, sl] + b_s[r, sl]) * d
            return carry

        lax.fori_loop(0, LANE, srow, 0)

    def emit_chunk(r0):
        def srow(r, carry):
            e = _splat(e_v, r0 + r)
            f = _splat(f_v, r0 + r)
            for i in range(D_Q // 16):
                sl = pl.ds(i * 16, 16)
                b_s[r, sl] = (a_s[r, sl] + f * b_s[r, sl]) * e
            return carry

        lax.fori_loop(0, LANE, srow, 0)

    for q in range(2):
        qbase = (2 * c + q) * N_NODES

        for i in range(RPT // LANE):
            rr = row0 + i * LANE
            if i < 3:
                pltpu.sync_copy(a_hbm.at[pl.ds(qbase + rr, LANE)], a_s)
                pltpu.sync_copy(b_hbm.at[pl.ds(qbase + rr, LANE)], b_s)
            elif i == 3:
                @pl.when(s < NS - 1)
                def _():
                    pltpu.sync_copy(a_hbm.at[pl.ds(qbase + rr, LANE)], a_s)
                    pltpu.sync_copy(b_hbm.at[pl.ds(qbase + rr, LANE)], b_s)

                @pl.when(s == NS - 1)
                def _():
                    pltpu.sync_copy(a_hbm.at[pl.ds(qbase + rr, 16)],
                                    a_s.at[pl.ds(0, 16)])
                    pltpu.sync_copy(b_hbm.at[pl.ds(qbase + rr, 16)],
                                    b_s.at[pl.ds(0, 16)])
            else:
                @pl.when(s < NS - 1)
                def _():
                    pltpu.sync_copy(a_hbm.at[pl.ds(qbase + rr, LANE)], a_s)
                    pltpu.sync_copy(b_hbm.at[pl.ds(qbase + rr, LANE)], b_s)
            stage_chunk(i * LANE)
            pltpu.sync_copy(b_s, g_sh.at[pl.ds(rr, LANE)])
            pltpu.sync_copy(zbuf, acc_sh.at[pl.ds(rr, LANE)])
        plsc.subcore_barrier()

        edge_pipeline()
        plsc.subcore_barrier()

        for i in range(RPT // LANE):
            rr = row0 + i * LANE
            pltpu.sync_copy(acc_sh.at[pl.ds(rr, LANE)], a_s)
            pltpu.sync_copy(g_sh.at[pl.ds(rr, LANE)], b_s)
            emit_chunk(i * LANE)
            if i < 3:
                pltpu.sync_copy(b_s, out_hbm.at[pl.ds(qbase + rr, LANE)])
            elif i == 3:
                @pl.when(s < NS - 1)
                def _():
                    pltpu.sync_copy(b_s,
                                    out_hbm.at[pl.ds(qbase + rr, LANE)])

                @pl.when(s == NS - 1)
                def _():
                    pltpu.sync_copy(b_s.at[pl.ds(0, 16)],
                                    out_hbm.at[pl.ds(qbase + rr, 16)])
            else:
                @pl.when(s < NS - 1)
                def _():
                    pltpu.sync_copy(b_s,
                                    out_hbm.at[pl.ds(qbase + rr, LANE)])

        plsc.subcore_barrier()


def _prop_call(args, t_rows):
    f = pl.kernel(
        _prop_body,
        out_type=jax.ShapeDtypeStruct((NQ * N_NODES, D_Q), jnp.float32),
        mesh=_MESH,
        scratch_types=[
            pltpu.VMEM((t_rows, LANE), jnp.int32),
            pltpu.VMEM((t_rows, LANE), jnp.int32),
            pltpu.VMEM((_NBUF, LANE, D_Q), jnp.float32),
            pltpu.VMEM((LANE, D_Q), jnp.float32),
            pltpu.VMEM((LANE, D_Q), jnp.float32),
            pltpu.VMEM((LANE, D_Q), jnp.float32),
            pltpu.VMEM((RPT,), jnp.float32),
            pltpu.VMEM((RPT,), jnp.float32),
            pltpu.VMEM((RPT,), jnp.float32),
            pltpu.SemaphoreType.DMA((_NBUF,)),
            pltpu.SemaphoreType.DMA((_NBUF,)),
            pltpu.VMEM_SHARED((N_PAD, D_Q), jnp.float32),
            pltpu.VMEM_SHARED((N_PAD, D_Q), jnp.float32),
        ],
        compiler_params=_SC_PARAMS,
    )
    return f(*args)


def _scale_x_body(x_ref, p_ref, g_ref, d_ref):
    deg = p_ref[0] + p_ref[1] + 1.0
    d = lax.rsqrt(deg)
    d_ref[...] = d
    for k in range(NQ):
        g_ref[k] = x_ref[:, k * D_Q:(k + 1) * D_Q] * d


def _scale_x_call(x, partials3):
    bn = 2000
    grid = N_NODES // bn
    return pl.pallas_call(
        _scale_x_body,
        grid=(grid,),
        in_specs=[
            pl.BlockSpec((bn, D_FEAT), lambda i: (i, 0)),
            pl.BlockSpec((NC, bn, 1), lambda i: (0, i, 0)),
        ],
        out_specs=[
            pl.BlockSpec((NQ, bn, D_Q), lambda i: (0, i, 0)),
            pl.BlockSpec((bn, 1), lambda i: (i, 0)),
        ],
        out_shape=[
            jax.ShapeDtypeStruct((NQ, N_NODES, D_Q), jnp.float32),
            jax.ShapeDtypeStruct((N_NODES, 1), jnp.float32),
        ],
    )(x, partials3)


def _final_body(h_ref, w_ref, b_ref, o_ref):
    h2 = jnp.concatenate([h_ref[k] for k in range(NQ)], axis=1)
    o = lax.dot_general(h2, w_ref[...],
                        dimension_numbers=(((1,), (1,)), ((), ())),
                        preferred_element_type=jnp.float32,
                        precision=lax.Precision.HIGHEST)
    o = o + b_ref[...]
    m = jnp.max(o, axis=1, keepdims=True)
    e = jnp.exp(o - m)
    lse = jnp.log(jnp.sum(e, axis=1, keepdims=True)) + m
    o_ref[...] = o - lse


def _final_call(h2, W, b2):
    bn = 2000
    grid = N_NODES // bn
    return pl.pallas_call(
        _final_body,
        grid=(grid,),
        in_specs=[
            pl.BlockSpec((NQ, bn, D_Q), lambda i: (0, i, 0)),
            pl.BlockSpec((N_CLASSES, D_FEAT), lambda i: (0, 0)),
            pl.BlockSpec((1, N_CLASSES), lambda i: (0, 0)),
        ],
        out_specs=pl.BlockSpec((bn, N_CLASSES), lambda i: (i, 0)),
        out_shape=jax.ShapeDtypeStruct((N_NODES, N_CLASSES), jnp.float32),
    )(h2, W, b2)


def kernel(x, edge_index, W, b):
    src = edge_index[0].astype(jnp.int32)
    dst = edge_index[1].astype(jnp.int32)
    e = src.shape[0]

    t1 = -(-e // (NC * NS * LANE))
    e1 = NC * NS * t1 * LANE
    dst_p1 = jnp.concatenate(
        [dst, jnp.full((e1 - e,), N_NODES, jnp.int32)]).reshape(
            NC * NS, t1, LANE)
    partials = _deg_call(dst_p1)

    partials3 = partials[:, :N_NODES].reshape(NC, N_NODES, 1)
    g0, dcol = _scale_x_call(x, partials3)

    t2 = -(-e // (NS * LANE))
    t2 = -(-t2 // _NBUF) * _NBUF
    e2 = NS * t2 * LANE
    src_p = jnp.concatenate(
        [src, jnp.zeros((e2 - e,), jnp.int32)]).reshape(NS, t2, LANE)
    dst_p = jnp.concatenate(
        [dst, jnp.full((e2 - e,), N_NODES, jnp.int32)]).reshape(NS, t2, LANE)

    g0f = g0.reshape(NQ * N_NODES, D_Q)
    dinv_pad = jnp.pad(dcol.reshape(N_NODES), (0, N_PAD - N_NODES))
    halves = jnp.full((N_PAD,), 0.5, jnp.float32)
    ones_n = jnp.ones((N_PAD,), jnp.float32)
    zeros_n = jnp.zeros((N_PAD,), jnp.float32)
    s1 = _prop_call((g0f, g0f, halves, ones_n, zeros_n, src_p, dst_p), t2)
    h2 = _prop_call((s1, g0f, dinv_pad * dinv_pad, dinv_pad, ones_n,
                     src_p, dst_p), t2)

    return _final_call(h2.reshape(NQ, N_NODES, D_Q), W,
                       b.reshape(1, N_CLASSES))


# speedup vs baseline: 1.1281x; 1.1281x over previous
import jax
import jax.numpy as jnp
from jax import lax
from jax.experimental import pallas as pl
from jax.experimental.pallas import tpu as pltpu
from jax.experimental.pallas import tpu_sc as plsc

N_NODES = 10000
N_PAD = 10240
ROWS_PER_TILE = N_PAD // 16
TAIL_ROWS = N_NODES - 640 * 15
D_FEAT = 128
D_Q = 32
NQ = 4
N_CLASSES = 64
NC = 2
NS = 16
LANE = 128

_MESH = plsc.VectorSubcoreMesh(core_axis_name="c", subcore_axis_name="s")
_SC_PARAMS = pltpu.CompilerParams(use_tc_tiling_on_sc=False)


def _deg_body(dst_hbm, out_hbm, idx_v, ones_v, zeros_v, deg_sh):
    c = lax.axis_index("c")
    s = lax.axis_index("s")
    w = c * NS + s
    t_rows = dst_hbm.shape[1]

    for i in range(LANE // 16):
        ones_v[pl.ds(i * 16, 16)] = jnp.ones((16,), jnp.float32)
    for i in range(ROWS_PER_TILE // 16):
        zeros_v[pl.ds(i * 16, 16)] = jnp.zeros((16,), jnp.float32)

    pltpu.sync_copy(zeros_v, deg_sh.at[pl.ds(s * ROWS_PER_TILE, ROWS_PER_TILE)])

    pltpu.sync_copy(dst_hbm.at[w], idx_v)
    plsc.subcore_barrier()

    def body(j, carry):
        pltpu.sync_copy(ones_v, deg_sh.at[idx_v.at[j]], add=True)
        return carry

    lax.fori_loop(0, t_rows, body, 0)
    plsc.subcore_barrier()

    pltpu.sync_copy(deg_sh.at[pl.ds(s * ROWS_PER_TILE, ROWS_PER_TILE)],
                    out_hbm.at[c, pl.ds(s * ROWS_PER_TILE, ROWS_PER_TILE)])


def _deg_call(dst_p):
    t_rows = dst_p.shape[1]
    f = pl.kernel(
        _deg_body,
        out_type=jax.ShapeDtypeStruct((NC, N_PAD), jnp.float32),
        mesh=_MESH,
        scratch_types=[
            pltpu.VMEM((t_rows, LANE), jnp.int32),
            pltpu.VMEM((LANE,), jnp.float32),
            pltpu.VMEM((ROWS_PER_TILE,), jnp.float32),
            pltpu.VMEM_SHARED((N_PAD,), jnp.float32),
        ],
        compiler_params=_SC_PARAMS,
    )
    return f(dst_p)


_NBUF = 4


def _prop_body(g_hbm, src_hbm, dst_hbm, out_hbm,
               sidx, didx, rows_v, zbuf, gsems, ssems, g_sh, acc_sh):
    c = lax.axis_index("c")
    s = lax.axis_index("s")
    t_rows = src_hbm.shape[1]
    n_groups = t_rows // _NBUF

    def zrow(r, carry):
        for i in range(D_Q // 16):
            zbuf[r, pl.ds(i * 16, 16)] = jnp.zeros((16,), jnp.float32)
        return carry

    lax.fori_loop(0, LANE, zrow, 0)

    pltpu.sync_copy(src_hbm.at[s], sidx)
    pltpu.sync_copy(dst_hbm.at[s], didx)

    def g_issue(j, b):
        pltpu.async_copy(g_sh.at[sidx.at[j]], rows_v.at[b], gsems.at[b])

    def g_wait(j, b):
        pltpu.make_async_copy(
            g_sh.at[sidx.at[j]], rows_v.at[b], gsems.at[b]).wait()

    def s_issue(j, b):
        pltpu.async_copy(rows_v.at[b], acc_sh.at[didx.at[j]], ssems.at[b],
                         add=True)

    def s_wait(j, b):
        pltpu.make_async_copy(
            rows_v.at[b], acc_sh.at[didx.at[j]], ssems.at[b]).wait()

    for q in range(2):
        qbase = (2 * c + q) * N_NODES

        @pl.when(s < NS - 1)
        def _():
            pltpu.sync_copy(
                g_hbm.at[pl.ds(qbase + s * ROWS_PER_TILE, ROWS_PER_TILE)],
                g_sh.at[pl.ds(s * ROWS_PER_TILE, ROWS_PER_TILE)])

        @pl.when(s == NS - 1)
        def _():
            pltpu.sync_copy(
                g_hbm.at[pl.ds(qbase + (NS - 1) * ROWS_PER_TILE, TAIL_ROWS)],
                g_sh.at[pl.ds((NS - 1) * ROWS_PER_TILE, TAIL_ROWS)])

        for i in range(ROWS_PER_TILE // LANE):
            pltpu.sync_copy(
                zbuf, acc_sh.at[pl.ds(s * ROWS_PER_TILE + i * LANE, LANE)])

        plsc.subcore_barrier()

        for b in range(_NBUF):
            g_issue(b, b)

        def group(g, carry):
            for b in range(_NBUF):
                j = g * _NBUF + b
                g_wait(j, b)
                s_issue(j, b)
                pb = (b - 1) % _NBUF
                if b == 0:
                    @pl.when(g > 0)
                    def _():
                        s_wait(j - 1, pb)
                        g_issue(j - 1 + _NBUF, pb)
                else:
                    s_wait(j - 1, pb)

                    @pl.when(g < n_groups - 1)
                    def _():
                        g_issue(j - 1 + _NBUF, pb)
            return carry

        lax.fori_loop(0, n_groups, group, 0)
        s_wait(t_rows - 1, (t_rows - 1) % _NBUF)
        plsc.subcore_barrier()

        @pl.when(s < NS - 1)
        def _():
            pltpu.sync_copy(
                acc_sh.at[pl.ds(s * ROWS_PER_TILE, ROWS_PER_TILE)],
                out_hbm.at[pl.ds(qbase + s * ROWS_PER_TILE, ROWS_PER_TILE)])

        @pl.when(s == NS - 1)
        def _():
            pltpu.sync_copy(
                acc_sh.at[pl.ds((NS - 1) * ROWS_PER_TILE, TAIL_ROWS)],
                out_hbm.at[pl.ds(qbase + (NS - 1) * ROWS_PER_TILE,
                                 TAIL_ROWS)])


def _prop_call(g_flat, src_p, dst_p):
    t_rows = src_p.shape[1]
    f = pl.kernel(
        _prop_body,
        out_type=jax.ShapeDtypeStruct((NQ * N_NODES, D_Q), jnp.float32),
        mesh=_MESH,
        scratch_types=[
            pltpu.VMEM((t_rows, LANE), jnp.int32),
            pltpu.VMEM((t_rows, LANE), jnp.int32),
            pltpu.VMEM((_NBUF, LANE, D_Q), jnp.float32),
            pltpu.VMEM((LANE, D_Q), jnp.float32),
            pltpu.SemaphoreType.DMA((_NBUF,)),
            pltpu.SemaphoreType.DMA((_NBUF,)),
            pltpu.VMEM_SHARED((N_PAD, D_Q), jnp.float32),
            pltpu.VMEM_SHARED((N_PAD, D_Q), jnp.float32),
        ],
        compiler_params=_SC_PARAMS,
    )
    return f(g_flat, src_p, dst_p)


def _scale_x_body(x_ref, p_ref, g_ref, d_ref):
    deg = p_ref[0] + p_ref[1] + 1.0
    d = lax.rsqrt(deg)
    d_ref[...] = d
    for k in range(NQ):
        g_ref[k] = x_ref[:, k * D_Q:(k + 1) * D_Q] * d


def _scale_x_call(x, partials3):
    bn = 2000
    grid = N_NODES // bn
    return pl.pallas_call(
        _scale_x_body,
        grid=(grid,),
        in_specs=[
            pl.BlockSpec((bn, D_FEAT), lambda i: (i, 0)),
            pl.BlockSpec((NC, bn, 1), lambda i: (0, i, 0)),
        ],
        out_specs=[
            pl.BlockSpec((NQ, bn, D_Q), lambda i: (0, i, 0)),
            pl.BlockSpec((bn, 1), lambda i: (i, 0)),
        ],
        out_shape=[
            jax.ShapeDtypeStruct((NQ, N_NODES, D_Q), jnp.float32),
            jax.ShapeDtypeStruct((N_NODES, 1), jnp.float32),
        ],
    )(x, partials3)


def _mid_body(s_ref, g_ref, d_ref, o_ref):
    d = d_ref[...]
    o_ref[...] = (s_ref[...] + g_ref[...]) * (d * d)


def _mid_call(s1, g0, dcol):
    bn = 2000
    grid = N_NODES // bn
    return pl.pallas_call(
        _mid_body,
        grid=(grid,),
        in_specs=[
            pl.BlockSpec((NQ, bn, D_Q), lambda i: (0, i, 0)),
            pl.BlockSpec((NQ, bn, D_Q), lambda i: (0, i, 0)),
            pl.BlockSpec((bn, 1), lambda i: (i, 0)),
        ],
        out_specs=pl.BlockSpec((NQ, bn, D_Q), lambda i: (0, i, 0)),
        out_shape=jax.ShapeDtypeStruct((NQ, N_NODES, D_Q), jnp.float32),
    )(s1, g0, dcol)


def _final_body(s_ref, g_ref, d_ref, w_ref, b_ref, o_ref):
    d = d_ref[...]
    h2 = jnp.concatenate(
        [(s_ref[k] + g_ref[k]) * d for k in range(NQ)], axis=1)
    o = lax.dot_general(h2, w_ref[...],
                        dimension_numbers=(((1,), (1,)), ((), ())),
                        preferred_element_type=jnp.float32,
                        precision=lax.Precision.HIGHEST)
    o = o + b_ref[...]
    m = jnp.max(o, axis=1, keepdims=True)
    e = jnp.exp(o - m)
    lse = jnp.log(jnp.sum(e, axis=1, keepdims=True)) + m
    o_ref[...] = o - lse


def _final_call(s2, g1, dcol, W, b2):
    bn = 2000
    grid = N_NODES // bn
    return pl.pallas_call(
        _final_body,
        grid=(grid,),
        in_specs=[
            pl.BlockSpec((NQ, bn, D_Q), lambda i: (0, i, 0)),
            pl.BlockSpec((NQ, bn, D_Q), lambda i: (0, i, 0)),
            pl.BlockSpec((bn, 1), lambda i: (i, 0)),
            pl.BlockSpec((N_CLASSES, D_FEAT), lambda i: (0, 0)),
            pl.BlockSpec((1, N_CLASSES), lambda i: (0, 0)),
        ],
        out_specs=pl.BlockSpec((bn, N_CLASSES), lambda i: (i, 0)),
        out_shape=jax.ShapeDtypeStruct((N_NODES, N_CLASSES), jnp.float32),
    )(s2, g1, dcol, W, b2)


def kernel(x, edge_index, W, b):
    src = edge_index[0].astype(jnp.int32)
    dst = edge_index[1].astype(jnp.int32)
    e = src.shape[0]

    t1 = -(-e // (NC * NS * LANE))
    e1 = NC * NS * t1 * LANE
    dst_p1 = jnp.concatenate(
        [dst, jnp.full((e1 - e,), N_NODES, jnp.int32)]).reshape(
            NC * NS, t1, LANE)
    partials = _deg_call(dst_p1)

    partials3 = partials[:, :N_NODES].reshape(NC, N_NODES, 1)
    g0, dcol = _scale_x_call(x, partials3)

    t2 = -(-e // (NS * LANE))
    t2 = -(-t2 // _NBUF) * _NBUF
    e2 = NS * t2 * LANE
    src_p = jnp.concatenate(
        [src, jnp.zeros((e2 - e,), jnp.int32)]).reshape(NS, t2, LANE)
    dst_p = jnp.concatenate(
        [dst, jnp.full((e2 - e,), N_NODES, jnp.int32)]).reshape(NS, t2, LANE)

    s1 = _prop_call(g0.reshape(NQ * N_NODES, D_Q), src_p, dst_p)
    g1 = _mid_call(s1.reshape(NQ, N_NODES, D_Q), g0, dcol)
    s2 = _prop_call(g1.reshape(NQ * N_NODES, D_Q), src_p, dst_p)

    return _final_call(s2.reshape(NQ, N_NODES, D_Q), g1, dcol, W,
                       b.reshape(1, N_CLASSES))
